# baseline (speedup 1.0000x reference)
import dataclasses
import functools

import jax
import jax.numpy as jnp
from jax import lax
from jax.experimental import pallas as pl
from jax.experimental.pallas import tpu as pltpu
from jax.experimental.pallas import tpu_sc as plsc

N = 10000
NPAD = 10240
D = 128
E = 320000
G = 64
OUT = 5

NW = 32
CHUNK = 96
GRP = 7
NGRP = 15
CPT = GRP * NGRP
EPT = CPT * CHUNK
EPAD = NW * EPT
RPS = NPAD // 16
NBLK = NPAD // 128

_mesh = plsc.VectorSubcoreMesh(core_axis_name="c", subcore_axis_name="s")

_sc_params = pltpu.CompilerParams()
if "needs_layout_passes" in pltpu.CompilerParams.__dataclass_fields__:
    _sc_params = dataclasses.replace(_sc_params, needs_layout_passes=False)

_f32 = jnp.float32
_DOT = dict(preferred_element_type=jnp.float32,
            precision=jax.lax.Precision.HIGHEST)


@functools.partial(
    pl.kernel,
    mesh=_mesh,
    compiler_params=_sc_params,
    out_type=[
        jax.ShapeDtypeStruct((NPAD, D), _f32),
        jax.ShapeDtypeStruct((NPAD, D), _f32),
        jax.ShapeDtypeStruct((NPAD,), _f32),
        jax.ShapeDtypeStruct((NPAD,), _f32),
    ],
    scratch_types=[
        pltpu.VMEM((NPAD,), _f32),
        pltpu.VMEM((NPAD,), _f32),
        pltpu.VMEM((GRP, CHUNK), jnp.int32),
        pltpu.VMEM((GRP, CHUNK), jnp.int32),
        pltpu.VMEM((GRP, CHUNK), _f32),
        pltpu.VMEM((CHUNK, D), _f32),
        pltpu.VMEM((CHUNK, D), _f32),
        pltpu.SemaphoreType.DMA,
        pltpu.SemaphoreType.DMA,
        pltpu.SemaphoreType.DMA,
        pltpu.SemaphoreType.DMA,
        pltpu.SemaphoreType.DMA,
        pltpu.VMEM_SHARED((NPAD, D), _f32),
        pltpu.VMEM_SHARED((NPAD,), _f32),
    ],
)
def _sc_edges(h_hbm, s_hbm, d_hbm, src_hbm, dst_hbm,
              accA, accB, denA, denB,
              sv, dv, srcg, dstg, wg, rows0, rows1,
              sem_w, sem_g0, sem_g1,
              sem_a0, sem_a1, accS, denS):
    c = lax.axis_index("c")
    s = lax.axis_index("s")
    wid = c * 16 + s
    rows = (rows0, rows1)
    sem_g = (sem_g0, sem_g1)
    sem_a = (sem_a0, sem_a1)

    @pl.loop(0, CHUNK)
    def _(r):
        for j in range(D // 16):
            rows0[r, pl.ds(j * 16, 16)] = jnp.zeros((16,), _f32)

    base = s * RPS
    for k in range(RPS // CHUNK):
        pltpu.sync_copy(rows0, accS.at[pl.ds(base + k * CHUNK, CHUNK)])
    rem = RPS % CHUNK
    if rem:
        pltpu.sync_copy(rows0.at[pl.ds(0, rem)],
                        accS.at[pl.ds(base + RPS - rem, rem)])
    for k in range(RPS // 128):
        pltpu.sync_copy(rows0.at[0], denS.at[pl.ds(base + k * 128, 128)])

    pltpu.sync_copy(s_hbm, sv)
    pltpu.sync_copy(d_hbm, dv)
    plsc.subcore_barrier()

    @pl.loop(0, NGRP)
    def _(g):
        pltpu.sync_copy(src_hbm.at[wid, g], srcg)
        pltpu.sync_copy(dst_hbm.at[wid, g], dstg)

        def fire(j, b):
            return pltpu.async_copy(h_hbm.at[srcg.at[j]], rows[b],
                                    sem_g[b])

        inflight = fire(0, 0)
        scat = [None, None]
        den_handles = []
        for j in range(GRP):
            b = j & 1
            inflight.wait()
            if j + 1 < GRP:
                if scat[1 - b] is not None:
                    scat[1 - b].wait()
                    scat[1 - b] = None
                inflight = fire(j + 1, 1 - b)
            for i in range(CHUNK // 16):
                sl = pl.ds(i * 16, 16)
                sidx = srcg[j, sl]
                didx = dstg[j, sl]
                e = (plsc.load_gather(sv, [sidx])
                     + plsc.load_gather(dv, [didx]))
                wg[j, sl] = jnp.exp(jnp.maximum(e, 0.2 * e))
            den_handles.append(
                pltpu.async_copy(wg.at[j], denS.at[dstg.at[j]], sem_w,
                                 add=True))

            @pl.loop(0, CHUNK // 16)
            def _(i):
                wv = wg[j, pl.ds(i * 16, 16)]
                for l in range(16):
                    w = wv[l]
                    row = i * 16 + l
                    for jj in range(D // 16):
                        rows[b][row, pl.ds(jj * 16, 16)] = (
                            rows[b][row, pl.ds(jj * 16, 16)] * w)

            scat[b] = pltpu.async_copy(rows[b], accS.at[dstg.at[j]],
                                       sem_a[b], add=True)
        for hd in scat:
            if hd is not None:
                hd.wait()
        for hd in den_handles:
            hd.wait()

    plsc.subcore_barrier()

    @pl.when(c == 0)
    def _():
        for k in range(RPS // CHUNK):
            sl = pl.ds(s * RPS + k * CHUNK, CHUNK)
            pltpu.sync_copy(accS.at[sl], accA.at[sl])
        pltpu.sync_copy(denS.at[pl.ds(s * RPS, RPS)],
                        denA.at[pl.ds(s * RPS, RPS)])

    @pl.when(c == 1)
    def _():
        for k in range(RPS // CHUNK):
            sl = pl.ds(s * RPS + k * CHUNK, CHUNK)
            pltpu.sync_copy(accS.at[sl], accB.at[sl])
        pltpu.sync_copy(denS.at[pl.ds(s * RPS, RPS)],
                        denB.at[pl.ds(s * RPS, RPS)])


def _dense_body(x_ref, w_ref, as_ref, ad_ref, h_ref, s_ref, d_ref, ws_ref):
    h = lax.dot_general(x_ref[...], w_ref[...], (((1,), (0,)), ((), ())),
                        **_DOT)
    sv = lax.dot_general(h, as_ref[...], (((1,), (0,)), ((), ())), **_DOT)
    dv = lax.dot_general(h, ad_ref[...], (((1,), (0,)), ((), ())), **_DOT)
    t = sv + dv
    h_ref[...] = h
    s_ref[...] = sv
    d_ref[...] = dv
    ws_ref[...] = jnp.exp(jnp.maximum(t, 0.2 * t))


_k_dense = pl.pallas_call(
    _dense_body,
    grid=(NBLK,),
    in_specs=[
        pl.BlockSpec((128, D), lambda i: (i, 0)),
        pl.BlockSpec((D, D), lambda i: (0, 0)),
        pl.BlockSpec((D, 1), lambda i: (0, 0)),
        pl.BlockSpec((D, 1), lambda i: (0, 0)),
    ],
    out_specs=[
        pl.BlockSpec((128, D), lambda i: (i, 0)),
        pl.BlockSpec((128, 1), lambda i: (i, 0)),
        pl.BlockSpec((128, 1), lambda i: (i, 0)),
        pl.BlockSpec((128, 1), lambda i: (i, 0)),
    ],
    out_shape=[
        jax.ShapeDtypeStruct((NPAD, D), _f32),
        jax.ShapeDtypeStruct((NPAD, 1), _f32),
        jax.ShapeDtypeStruct((NPAD, 1), _f32),
        jax.ShapeDtypeStruct((NPAD, 1), _f32),
    ],
)


def _norm(accA, accB, h, ws, dA, dB, b):
    den = dA + dB + ws + 1e-16
    acc = accA + accB + ws * h
    return jnp.maximum(acc / den + b, 0.0)


def _norm_dense_body(aA_ref, aB_ref, h_ref, ws_ref, dA_ref, dB_ref, b_ref,
                     w_ref, as_ref, ad_ref,
                     h2_ref, s_ref, d_ref, ws2_ref):
    x2 = _norm(aA_ref[...], aB_ref[...], h_ref[...], ws_ref[...],
               dA_ref[...], dB_ref[...], b_ref[...])
    h2 = lax.dot_general(x2, w_ref[...], (((1,), (0,)), ((), ())), **_DOT)
    sv = lax.dot_general(h2, as_ref[...], (((1,), (0,)), ((), ())), **_DOT)
    dv = lax.dot_general(h2, ad_ref[...], (((1,), (0,)), ((), ())), **_DOT)
    t = sv + dv
    h2_ref[...] = h2
    s_ref[...] = sv
    d_ref[...] = dv
    ws2_ref[...] = jnp.exp(jnp.maximum(t, 0.2 * t))


_k_norm_dense = pl.pallas_call(
    _norm_dense_body,
    grid=(NBLK,),
    in_specs=[
        pl.BlockSpec((128, D), lambda i: (i, 0)),
        pl.BlockSpec((128, D), lambda i: (i, 0)),
        pl.BlockSpec((128, D), lambda i: (i, 0)),
        pl.BlockSpec((128, 1), lambda i: (i, 0)),
        pl.BlockSpec((128, 1), lambda i: (i, 0)),
        pl.BlockSpec((128, 1), lambda i: (i, 0)),
        pl.BlockSpec((1, D), lambda i: (0, 0)),
        pl.BlockSpec((D, D), lambda i: (0, 0)),
        pl.BlockSpec((D, 1), lambda i: (0, 0)),
        pl.BlockSpec((D, 1), lambda i: (0, 0)),
    ],
    out_specs=[
        pl.BlockSpec((128, D), lambda i: (i, 0)),
        pl.BlockSpec((128, 1), lambda i: (i, 0)),
        pl.BlockSpec((128, 1), lambda i: (i, 0)),
        pl.BlockSpec((128, 1), lambda i: (i, 0)),
    ],
    out_shape=[
        jax.ShapeDtypeStruct((NPAD, D), _f32),
        jax.ShapeDtypeStruct((NPAD, 1), _f32),
        jax.ShapeDtypeStruct((NPAD, 1), _f32),
        jax.ShapeDtypeStruct((NPAD, 1), _f32),
    ],
)


def _final_body(aA_ref, aB_ref, h_ref, ws_ref, dA_ref, dB_ref, b_ref,
                batch_ref, wl_ref, bl_ref, o_ref, pooled_ref):
    i = pl.program_id(0)
    hout = _norm(aA_ref[...], aB_ref[...], h_ref[...], ws_ref[...],
                 dA_ref[...], dB_ref[...], b_ref[...])
    bcol = batch_ref[0]

    @pl.when(i == 0)
    def _():
        pooled_ref[...] = jnp.zeros((G, D), _f32)

    gmin = jnp.min(bcol)
    gmax = jnp.minimum(jnp.max(bcol), G - 1)

    def _pool_one(g, carry):
        red = jnp.max(jnp.where(bcol == g, hout, 0.0), axis=0, keepdims=True)
        pooled_ref[pl.ds(g, 1), :] = jnp.maximum(pooled_ref[pl.ds(g, 1), :],
                                                 red)
        return carry

    lax.fori_loop(gmin, gmax + 1, _pool_one, 0)

    @pl.when(i == NBLK - 1)
    def _():
        o_ref[...] = (lax.dot_general(pooled_ref[...], wl_ref[...],
                                      (((1,), (0,)), ((), ())), **_DOT)
                      + bl_ref[...])


_k_final = pl.pallas_call(
    _final_body,
    grid=(NBLK,),
    in_specs=[
        pl.BlockSpec((128, D), lambda i: (i, 0)),
        pl.BlockSpec((128, D), lambda i: (i, 0)),
        pl.BlockSpec((128, D), lambda i: (i, 0)),
        pl.BlockSpec((128, 1), lambda i: (i, 0)),
        pl.BlockSpec((128, 1), lambda i: (i, 0)),
        pl.BlockSpec((128, 1), lambda i: (i, 0)),
        pl.BlockSpec((1, D), lambda i: (0, 0)),
        pl.BlockSpec((1, 128, 1), lambda i: (i, 0, 0)),
        pl.BlockSpec((D, 128), lambda i: (0, 0)),
        pl.BlockSpec((1, 128), lambda i: (0, 0)),
    ],
    out_specs=[pl.BlockSpec((G, 128), lambda i: (0, 0))],
    out_shape=[jax.ShapeDtypeStruct((G, 128), _f32)],
    scratch_shapes=[pltpu.VMEM((G, D), _f32)],
)


def kernel(x, edge_index, batch, W1, a_s1, a_d1, b1, W2, a_s2, a_d2, b2,
           Wl, bl):
    src = edge_index[0]
    dst = edge_index[1]
    pad = N + (jnp.arange(EPAD - E, dtype=jnp.int32) % (NPAD - N))
    srcp = jnp.concatenate([src, pad]).reshape(NW, NGRP, GRP, CHUNK)
    dstp = jnp.concatenate([dst, pad]).reshape(NW, NGRP, GRP, CHUNK)
    xp = jnp.zeros((NPAD, D), _f32).at[:N].set(x)
    batchp = jnp.concatenate(
        [batch, jnp.full((NPAD - N,), G, jnp.int32)]).reshape(NBLK, 128, 1)
    wlp = jnp.zeros((D, 128), _f32).at[:, :OUT].set(Wl)
    blp = jnp.zeros((1, 128), _f32).at[0, :OUT].set(bl)

    h1, s1, d1, ws1 = _k_dense(xp, W1, a_s1.reshape(D, 1),
                               a_d1.reshape(D, 1))
    accA, accB, denA, denB = _sc_edges(
        h1, s1.reshape(NPAD), d1.reshape(NPAD), srcp, dstp)
    h2, s2, d2, ws2 = _k_norm_dense(
        accA, accB, h1, ws1, denA.reshape(NPAD, 1), denB.reshape(NPAD, 1),
        b1.reshape(1, D), W2, a_s2.reshape(D, 1), a_d2.reshape(D, 1))
    accA2, accB2, denA2, denB2 = _sc_edges(
        h2, s2.reshape(NPAD), d2.reshape(NPAD), srcp, dstp)
    o = _k_final(accA2, accB2, h2, ws2, denA2.reshape(NPAD, 1),
                 denB2.reshape(NPAD, 1), b2.reshape(1, D), batchp, wlp, blp)
    return o[0][:, :OUT]


# speedup vs baseline: 1.0575x; 1.0575x over previous
import dataclasses
import functools

import jax
import jax.numpy as jnp
from jax import lax
from jax.experimental import pallas as pl
from jax.experimental.pallas import tpu as pltpu
from jax.experimental.pallas import tpu_sc as plsc

N = 10000
NPAD = 10240
D = 128
E = 320000
G = 64
OUT = 5

NW = 32
CHUNK = 128
GRP = 8
NGRP = 10
CPT = GRP * NGRP
EPT = CPT * CHUNK
EPAD = NW * EPT
RPS = NPAD // 16
NBLK = NPAD // 128

_mesh = plsc.VectorSubcoreMesh(core_axis_name="c", subcore_axis_name="s")

_sc_params = pltpu.CompilerParams()
if "needs_layout_passes" in pltpu.CompilerParams.__dataclass_fields__:
    _sc_params = dataclasses.replace(_sc_params, needs_layout_passes=False)

_f32 = jnp.float32
_DOT = dict(preferred_element_type=jnp.float32,
            precision=jax.lax.Precision.HIGHEST)


@functools.partial(
    pl.kernel,
    mesh=_mesh,
    compiler_params=_sc_params,
    out_type=[
        jax.ShapeDtypeStruct((NPAD, D), _f32),
        jax.ShapeDtypeStruct((NPAD, D), _f32),
        jax.ShapeDtypeStruct((NPAD,), _f32),
        jax.ShapeDtypeStruct((NPAD,), _f32),
    ],
    scratch_types=[
        pltpu.VMEM((NPAD,), jnp.int32),
        pltpu.VMEM((GRP, CHUNK), jnp.int32),
        pltpu.VMEM((GRP, CHUNK), jnp.int32),
        pltpu.VMEM((GRP, CHUNK), _f32),
        pltpu.VMEM((CHUNK, D), _f32),
        pltpu.VMEM((CHUNK, D), _f32),
        pltpu.SemaphoreType.DMA,
        pltpu.SemaphoreType.DMA,
        pltpu.SemaphoreType.DMA,
        pltpu.SemaphoreType.DMA,
        pltpu.SemaphoreType.DMA,
        pltpu.VMEM_SHARED((NPAD, D), _f32),
        pltpu.VMEM_SHARED((NPAD,), _f32),
    ],
)
def _sc_edges(h_hbm, t_hbm, src_hbm, dst_hbm,
              accA, accB, denA, denB,
              tv, srcg, dstg, wg, rows0, rows1,
              sem_w, sem_g0, sem_g1,
              sem_a0, sem_a1, accS, denS):
    c = lax.axis_index("c")
    s = lax.axis_index("s")
    wid = c * 16 + s
    rows = (rows0, rows1)
    sem_g = (sem_g0, sem_g1)
    sem_a = (sem_a0, sem_a1)

    @pl.loop(0, CHUNK)
    def _(r):
        for j in range(D // 16):
            rows0[r, pl.ds(j * 16, 16)] = jnp.zeros((16,), _f32)

    base = s * RPS
    for k in range(RPS // CHUNK):
        pltpu.sync_copy(rows0, accS.at[pl.ds(base + k * CHUNK, CHUNK)])
    rem = RPS % CHUNK
    if rem:
        pltpu.sync_copy(rows0.at[pl.ds(0, rem)],
                        accS.at[pl.ds(base + RPS - rem, rem)])
    for k in range(RPS // 128):
        pltpu.sync_copy(rows0.at[0], denS.at[pl.ds(base + k * 128, 128)])

    pltpu.sync_copy(t_hbm, tv)
    plsc.subcore_barrier()

    @pl.loop(0, NGRP)
    def _(g):
        pltpu.sync_copy(src_hbm.at[wid, g], srcg)
        pltpu.sync_copy(dst_hbm.at[wid, g], dstg)

        def fire(j, b):
            return pltpu.async_copy(h_hbm.at[srcg.at[j]], rows[b],
                                    sem_g[b])

        inflight = fire(0, 0)
        scat = [None, None]
        den_handles = []
        for j in range(GRP):
            b = j & 1
            inflight.wait()
            if j + 1 < GRP:
                if scat[1 - b] is not None:
                    scat[1 - b].wait()
                    scat[1 - b] = None
                inflight = fire(j + 1, 1 - b)
            for i in range(CHUNK // 16):
                sl = pl.ds(i * 16, 16)
                gs = plsc.load_gather(tv, [srcg[j, sl]])
                gd = plsc.load_gather(tv, [dstg[j, sl]])
                sval = plsc.bitcast(gs & jnp.int32(-65536), _f32)
                dval = plsc.bitcast(lax.shift_left(gd, 16), _f32)
                e = sval + dval
                wg[j, sl] = jnp.exp(jnp.maximum(e, 0.2 * e))
            den_handles.append(
                pltpu.async_copy(wg.at[j], denS.at[dstg.at[j]], sem_w,
                                 add=True))

            @pl.loop(0, CHUNK // 16)
            def _(i):
                wv = wg[j, pl.ds(i * 16, 16)]
                for l in range(16):
                    w = wv[l]
                    row = i * 16 + l
                    for jj in range(D // 16):
                        rows[b][row, pl.ds(jj * 16, 16)] = (
                            rows[b][row, pl.ds(jj * 16, 16)] * w)

            scat[b] = pltpu.async_copy(rows[b], accS.at[dstg.at[j]],
                                       sem_a[b], add=True)
        for hd in scat:
            if hd is not None:
                hd.wait()
        for hd in den_handles:
            hd.wait()

    plsc.subcore_barrier()

    @pl.when(c == 0)
    def _():
        for k in range(RPS // CHUNK):
            sl = pl.ds(s * RPS + k * CHUNK, CHUNK)
            pltpu.sync_copy(accS.at[sl], accA.at[sl])
        pltpu.sync_copy(denS.at[pl.ds(s * RPS, RPS)],
                        denA.at[pl.ds(s * RPS, RPS)])

    @pl.when(c == 1)
    def _():
        for k in range(RPS // CHUNK):
            sl = pl.ds(s * RPS + k * CHUNK, CHUNK)
            pltpu.sync_copy(accS.at[sl], accB.at[sl])
        pltpu.sync_copy(denS.at[pl.ds(s * RPS, RPS)],
                        denB.at[pl.ds(s * RPS, RPS)])


def _dense_body(x_ref, w_ref, as_ref, ad_ref, h_ref, s_ref, d_ref, ws_ref):
    h = lax.dot_general(x_ref[...], w_ref[...], (((1,), (0,)), ((), ())),
                        **_DOT)
    sv = lax.dot_general(h, as_ref[...], (((1,), (0,)), ((), ())), **_DOT)
    dv = lax.dot_general(h, ad_ref[...], (((1,), (0,)), ((), ())), **_DOT)
    t = sv + dv
    h_ref[...] = h
    s_ref[...] = sv
    d_ref[...] = dv
    ws_ref[...] = jnp.exp(jnp.maximum(t, 0.2 * t))


_k_dense = pl.pallas_call(
    _dense_body,
    grid=(NBLK,),
    in_specs=[
        pl.BlockSpec((128, D), lambda i: (i, 0)),
        pl.BlockSpec((D, D), lambda i: (0, 0)),
        pl.BlockSpec((D, 1), lambda i: (0, 0)),
        pl.BlockSpec((D, 1), lambda i: (0, 0)),
    ],
    out_specs=[
        pl.BlockSpec((128, D), lambda i: (i, 0)),
        pl.BlockSpec((128, 1), lambda i: (i, 0)),
        pl.BlockSpec((128, 1), lambda i: (i, 0)),
        pl.BlockSpec((128, 1), lambda i: (i, 0)),
    ],
    out_shape=[
        jax.ShapeDtypeStruct((NPAD, D), _f32),
        jax.ShapeDtypeStruct((NPAD, 1), _f32),
        jax.ShapeDtypeStruct((NPAD, 1), _f32),
        jax.ShapeDtypeStruct((NPAD, 1), _f32),
    ],
)


def _norm(accA, accB, h, ws, dA, dB, b):
    den = dA + dB + ws + 1e-16
    acc = accA + accB + ws * h
    return jnp.maximum(acc / den + b, 0.0)


def _norm_dense_body(aA_ref, aB_ref, h_ref, ws_ref, dA_ref, dB_ref, b_ref,
                     w_ref, as_ref, ad_ref,
                     h2_ref, s_ref, d_ref, ws2_ref):
    x2 = _norm(aA_ref[...], aB_ref[...], h_ref[...], ws_ref[...],
               dA_ref[...], dB_ref[...], b_ref[...])
    h2 = lax.dot_general(x2, w_ref[...], (((1,), (0,)), ((), ())), **_DOT)
    sv = lax.dot_general(h2, as_ref[...], (((1,), (0,)), ((), ())), **_DOT)
    dv = lax.dot_general(h2, ad_ref[...], (((1,), (0,)), ((), ())), **_DOT)
    t = sv + dv
    h2_ref[...] = h2
    s_ref[...] = sv
    d_ref[...] = dv
    ws2_ref[...] = jnp.exp(jnp.maximum(t, 0.2 * t))


_k_norm_dense = pl.pallas_call(
    _norm_dense_body,
    grid=(NBLK,),
    in_specs=[
        pl.BlockSpec((128, D), lambda i: (i, 0)),
        pl.BlockSpec((128, D), lambda i: (i, 0)),
        pl.BlockSpec((128, D), lambda i: (i, 0)),
        pl.BlockSpec((128, 1), lambda i: (i, 0)),
        pl.BlockSpec((128, 1), lambda i: (i, 0)),
        pl.BlockSpec((128, 1), lambda i: (i, 0)),
        pl.BlockSpec((1, D), lambda i: (0, 0)),
        pl.BlockSpec((D, D), lambda i: (0, 0)),
        pl.BlockSpec((D, 1), lambda i: (0, 0)),
        pl.BlockSpec((D, 1), lambda i: (0, 0)),
    ],
    out_specs=[
        pl.BlockSpec((128, D), lambda i: (i, 0)),
        pl.BlockSpec((128, 1), lambda i: (i, 0)),
        pl.BlockSpec((128, 1), lambda i: (i, 0)),
        pl.BlockSpec((128, 1), lambda i: (i, 0)),
    ],
    out_shape=[
        jax.ShapeDtypeStruct((NPAD, D), _f32),
        jax.ShapeDtypeStruct((NPAD, 1), _f32),
        jax.ShapeDtypeStruct((NPAD, 1), _f32),
        jax.ShapeDtypeStruct((NPAD, 1), _f32),
    ],
)


def _final_body(aA_ref, aB_ref, h_ref, ws_ref, dA_ref, dB_ref, b_ref,
                batch_ref, wl_ref, bl_ref, o_ref, pooled_ref):
    i = pl.program_id(0)
    hout = _norm(aA_ref[...], aB_ref[...], h_ref[...], ws_ref[...],
                 dA_ref[...], dB_ref[...], b_ref[...])
    bcol = batch_ref[0]

    @pl.when(i == 0)
    def _():
        pooled_ref[...] = jnp.zeros((G, D), _f32)

    gmin = jnp.min(bcol)
    gmax = jnp.minimum(jnp.max(bcol), G - 1)

    def _pool_one(g, carry):
        red = jnp.max(jnp.where(bcol == g, hout, 0.0), axis=0, keepdims=True)
        pooled_ref[pl.ds(g, 1), :] = jnp.maximum(pooled_ref[pl.ds(g, 1), :],
                                                 red)
        return carry

    lax.fori_loop(gmin, gmax + 1, _pool_one, 0)

    @pl.when(i == NBLK - 1)
    def _():
        o_ref[...] = (lax.dot_general(pooled_ref[...], wl_ref[...],
                                      (((1,), (0,)), ((), ())), **_DOT)
                      + bl_ref[...])


_k_final = pl.pallas_call(
    _final_body,
    grid=(NBLK,),
    in_specs=[
        pl.BlockSpec((128, D), lambda i: (i, 0)),
        pl.BlockSpec((128, D), lambda i: (i, 0)),
        pl.BlockSpec((128, D), lambda i: (i, 0)),
        pl.BlockSpec((128, 1), lambda i: (i, 0)),
        pl.BlockSpec((128, 1), lambda i: (i, 0)),
        pl.BlockSpec((128, 1), lambda i: (i, 0)),
        pl.BlockSpec((1, D), lambda i: (0, 0)),
        pl.BlockSpec((1, 128, 1), lambda i: (i, 0, 0)),
        pl.BlockSpec((D, 128), lambda i: (0, 0)),
        pl.BlockSpec((1, 128), lambda i: (0, 0)),
    ],
    out_specs=[pl.BlockSpec((G, 128), lambda i: (0, 0))],
    out_shape=[jax.ShapeDtypeStruct((G, 128), _f32)],
    scratch_shapes=[pltpu.VMEM((G, D), _f32)],
)


def kernel(x, edge_index, batch, W1, a_s1, a_d1, b1, W2, a_s2, a_d2, b2,
           Wl, bl):
    src = edge_index[0]
    dst = edge_index[1]
    pad = N + (jnp.arange(EPAD - E, dtype=jnp.int32) % (NPAD - N))
    srcp = jnp.concatenate([src, pad]).reshape(NW, NGRP, GRP, CHUNK)
    dstp = jnp.concatenate([dst, pad]).reshape(NW, NGRP, GRP, CHUNK)
    xp = jnp.zeros((NPAD, D), _f32).at[:N].set(x)
    batchp = jnp.concatenate(
        [batch, jnp.full((NPAD - N,), G, jnp.int32)]).reshape(NBLK, 128, 1)
    wlp = jnp.zeros((D, 128), _f32).at[:, :OUT].set(Wl)
    blp = jnp.zeros((1, 128), _f32).at[0, :OUT].set(bl)

    def pack_logits(sarr, darr):
        sb = jax.lax.bitcast_convert_type(
            sarr.reshape(NPAD).astype(jnp.bfloat16), jnp.uint16)
        db = jax.lax.bitcast_convert_type(
            darr.reshape(NPAD).astype(jnp.bfloat16), jnp.uint16)
        packed = (sb.astype(jnp.uint32) << 16) | db.astype(jnp.uint32)
        return jax.lax.bitcast_convert_type(packed, jnp.int32)

    h1, s1, d1, ws1 = _k_dense(xp, W1, a_s1.reshape(D, 1),
                               a_d1.reshape(D, 1))
    accA, accB, denA, denB = _sc_edges(h1, pack_logits(s1, d1), srcp, dstp)
    h2, s2, d2, ws2 = _k_norm_dense(
        accA, accB, h1, ws1, denA.reshape(NPAD, 1), denB.reshape(NPAD, 1),
        b1.reshape(1, D), W2, a_s2.reshape(D, 1), a_d2.reshape(D, 1))
    accA2, accB2, denA2, denB2 = _sc_edges(h2, pack_logits(s2, d2),
                                           srcp, dstp)
    o = _k_final(accA2, accB2, h2, ws2, denA2.reshape(NPAD, 1),
                 denB2.reshape(NPAD, 1), b2.reshape(1, D), batchp, wlp, blp)
    return o[0][:, :OUT]


# speedup vs baseline: 1.2729x; 1.2036x over previous
import dataclasses
import functools

import jax
import jax.numpy as jnp
from jax import lax
from jax.experimental import pallas as pl
from jax.experimental.pallas import tpu as pltpu
from jax.experimental.pallas import tpu_sc as plsc

N = 10000
NPAD = 10240
D = 128
E = 320000
G = 64
OUT = 5

NW = 32
CHUNK = 128
GRP = 8
NGRP = 10
CPT = GRP * NGRP
EPT = CPT * CHUNK
EPAD = NW * EPT
RPS = NPAD // 16
NBLK = NPAD // 128
BROW = 512
NBLKD = NPAD // BROW
PROW = 256
NBLKP = NPAD // PROW

_mesh = plsc.VectorSubcoreMesh(core_axis_name="c", subcore_axis_name="s")

_sc_params = pltpu.CompilerParams()
if "needs_layout_passes" in pltpu.CompilerParams.__dataclass_fields__:
    _sc_params = dataclasses.replace(_sc_params, needs_layout_passes=False)

_f32 = jnp.float32
_DOT = dict(preferred_element_type=jnp.float32,
            precision=jax.lax.Precision.HIGHEST)


@functools.partial(
    pl.kernel,
    mesh=_mesh,
    compiler_params=_sc_params,
    out_type=[
        jax.ShapeDtypeStruct((NPAD, D), _f32),
        jax.ShapeDtypeStruct((NPAD, D), _f32),
        jax.ShapeDtypeStruct((NPAD,), _f32),
        jax.ShapeDtypeStruct((NPAD,), _f32),
    ],
    scratch_types=[
        pltpu.VMEM((NPAD,), jnp.int32),
        pltpu.VMEM((GRP, CHUNK), jnp.int32),
        pltpu.VMEM((GRP, CHUNK), jnp.int32),
        pltpu.VMEM((GRP, CHUNK), _f32),
        pltpu.VMEM((CHUNK, D), _f32),
        pltpu.VMEM((CHUNK, D), _f32),
        pltpu.SemaphoreType.DMA,
        pltpu.SemaphoreType.DMA,
        pltpu.SemaphoreType.DMA,
        pltpu.SemaphoreType.DMA,
        pltpu.SemaphoreType.DMA,
        pltpu.VMEM_SHARED((NPAD, D), _f32),
        pltpu.VMEM_SHARED((NPAD,), _f32),
    ],
)
def _sc_edges(h_hbm, t_hbm, src_hbm, dst_hbm,
              accA, accB, denA, denB,
              tv, srcg, dstg, wg, rows0, rows1,
              sem_w, sem_g0, sem_g1,
              sem_a0, sem_a1, accS, denS):
    c = lax.axis_index("c")
    s = lax.axis_index("s")
    wid = c * 16 + s
    rows = (rows0, rows1)
    sem_g = (sem_g0, sem_g1)
    sem_a = (sem_a0, sem_a1)

    @pl.loop(0, CHUNK)
    def _(r):
        for j in range(D // 16):
            rows0[r, pl.ds(j * 16, 16)] = jnp.zeros((16,), _f32)

    base = s * RPS
    for k in range(RPS // CHUNK):
        pltpu.sync_copy(rows0, accS.at[pl.ds(base + k * CHUNK, CHUNK)])
    rem = RPS % CHUNK
    if rem:
        pltpu.sync_copy(rows0.at[pl.ds(0, rem)],
                        accS.at[pl.ds(base + RPS - rem, rem)])
    for k in range(RPS // 128):
        pltpu.sync_copy(rows0.at[0], denS.at[pl.ds(base + k * 128, 128)])

    pltpu.sync_copy(t_hbm, tv)
    plsc.subcore_barrier()

    @pl.loop(0, NGRP)
    def _(g):
        pltpu.sync_copy(src_hbm.at[wid, g], srcg)
        pltpu.sync_copy(dst_hbm.at[wid, g], dstg)

        def fire(j, b):
            return pltpu.async_copy(h_hbm.at[srcg.at[j]], rows[b],
                                    sem_g[b])

        inflight = fire(0, 0)
        scat = [None, None]
        den_handles = []
        for j in range(GRP):
            b = j & 1
            inflight.wait()
            if j + 1 < GRP:
                if scat[1 - b] is not None:
                    scat[1 - b].wait()
                    scat[1 - b] = None
                inflight = fire(j + 1, 1 - b)
            for i in range(CHUNK // 16):
                sl = pl.ds(i * 16, 16)
                gs = plsc.load_gather(tv, [srcg[j, sl]])
                gd = plsc.load_gather(tv, [dstg[j, sl]])
                sval = plsc.bitcast(gs & jnp.int32(-65536), _f32)
                dval = plsc.bitcast(lax.shift_left(gd, 16), _f32)
                e = sval + dval
                wg[j, sl] = jnp.exp(jnp.maximum(e, 0.2 * e))
            den_handles.append(
                pltpu.async_copy(wg.at[j], denS.at[dstg.at[j]], sem_w,
                                 add=True))

            @pl.loop(0, CHUNK // 16)
            def _(i):
                wv = wg[j, pl.ds(i * 16, 16)]
                for l in range(16):
                    w = wv[l]
                    row = i * 16 + l
                    for jj in range(D // 16):
                        rows[b][row, pl.ds(jj * 16, 16)] = (
                            rows[b][row, pl.ds(jj * 16, 16)] * w)

            scat[b] = pltpu.async_copy(rows[b], accS.at[dstg.at[j]],
                                       sem_a[b], add=True)
        for hd in scat:
            if hd is not None:
                hd.wait()
        for hd in den_handles:
            hd.wait()

    plsc.subcore_barrier()

    @pl.when(c == 0)
    def _():
        for k in range(RPS // CHUNK):
            sl = pl.ds(s * RPS + k * CHUNK, CHUNK)
            pltpu.sync_copy(accS.at[sl], accA.at[sl])
        pltpu.sync_copy(denS.at[pl.ds(s * RPS, RPS)],
                        denA.at[pl.ds(s * RPS, RPS)])

    @pl.when(c == 1)
    def _():
        for k in range(RPS // CHUNK):
            sl = pl.ds(s * RPS + k * CHUNK, CHUNK)
            pltpu.sync_copy(accS.at[sl], accB.at[sl])
        pltpu.sync_copy(denS.at[pl.ds(s * RPS, RPS)],
                        denB.at[pl.ds(s * RPS, RPS)])


def _dense_body(x_ref, w_ref, as_ref, ad_ref, h_ref, s_ref, d_ref, ws_ref):
    h = lax.dot_general(x_ref[...], w_ref[...], (((1,), (0,)), ((), ())),
                        **_DOT)
    sv = lax.dot_general(h, as_ref[...], (((1,), (0,)), ((), ())), **_DOT)
    dv = lax.dot_general(h, ad_ref[...], (((1,), (0,)), ((), ())), **_DOT)
    t = sv + dv
    h_ref[...] = h
    s_ref[...] = sv
    d_ref[...] = dv
    ws_ref[...] = jnp.exp(jnp.maximum(t, 0.2 * t))


_k_dense = pl.pallas_call(
    _dense_body,
    grid=(NBLKD,),
    in_specs=[
        pl.BlockSpec((BROW, D), lambda i: (i, 0)),
        pl.BlockSpec((D, D), lambda i: (0, 0)),
        pl.BlockSpec((D, 1), lambda i: (0, 0)),
        pl.BlockSpec((D, 1), lambda i: (0, 0)),
    ],
    out_specs=[
        pl.BlockSpec((BROW, D), lambda i: (i, 0)),
        pl.BlockSpec((BROW, 1), lambda i: (i, 0)),
        pl.BlockSpec((BROW, 1), lambda i: (i, 0)),
        pl.BlockSpec((BROW, 1), lambda i: (i, 0)),
    ],
    out_shape=[
        jax.ShapeDtypeStruct((NPAD, D), _f32),
        jax.ShapeDtypeStruct((NPAD, 1), _f32),
        jax.ShapeDtypeStruct((NPAD, 1), _f32),
        jax.ShapeDtypeStruct((NPAD, 1), _f32),
    ],
)


def _norm(accA, accB, h, ws, dA, dB, b):
    den = dA + dB + ws + 1e-16
    acc = accA + accB + ws * h
    return jnp.maximum(acc / den + b, 0.0)


def _norm_dense_body(aA_ref, aB_ref, h_ref, ws_ref, dA_ref, dB_ref, b_ref,
                     w_ref, as_ref, ad_ref,
                     h2_ref, s_ref, d_ref, ws2_ref):
    x2 = _norm(aA_ref[...], aB_ref[...], h_ref[...], ws_ref[...],
               dA_ref[...], dB_ref[...], b_ref[...])
    h2 = lax.dot_general(x2, w_ref[...], (((1,), (0,)), ((), ())), **_DOT)
    sv = lax.dot_general(h2, as_ref[...], (((1,), (0,)), ((), ())), **_DOT)
    dv = lax.dot_general(h2, ad_ref[...], (((1,), (0,)), ((), ())), **_DOT)
    t = sv + dv
    h2_ref[...] = h2
    s_ref[...] = sv
    d_ref[...] = dv
    ws2_ref[...] = jnp.exp(jnp.maximum(t, 0.2 * t))


_k_norm_dense = pl.pallas_call(
    _norm_dense_body,
    grid=(NBLKD,),
    in_specs=[
        pl.BlockSpec((BROW, D), lambda i: (i, 0)),
        pl.BlockSpec((BROW, D), lambda i: (i, 0)),
        pl.BlockSpec((BROW, D), lambda i: (i, 0)),
        pl.BlockSpec((BROW, 1), lambda i: (i, 0)),
        pl.BlockSpec((BROW, 1), lambda i: (i, 0)),
        pl.BlockSpec((BROW, 1), lambda i: (i, 0)),
        pl.BlockSpec((1, D), lambda i: (0, 0)),
        pl.BlockSpec((D, D), lambda i: (0, 0)),
        pl.BlockSpec((D, 1), lambda i: (0, 0)),
        pl.BlockSpec((D, 1), lambda i: (0, 0)),
    ],
    out_specs=[
        pl.BlockSpec((BROW, D), lambda i: (i, 0)),
        pl.BlockSpec((BROW, 1), lambda i: (i, 0)),
        pl.BlockSpec((BROW, 1), lambda i: (i, 0)),
        pl.BlockSpec((BROW, 1), lambda i: (i, 0)),
    ],
    out_shape=[
        jax.ShapeDtypeStruct((NPAD, D), _f32),
        jax.ShapeDtypeStruct((NPAD, 1), _f32),
        jax.ShapeDtypeStruct((NPAD, 1), _f32),
        jax.ShapeDtypeStruct((NPAD, 1), _f32),
    ],
)


def _final_body(aA_ref, aB_ref, h_ref, ws_ref, dA_ref, dB_ref, b_ref,
                batch_ref, wl_ref, bl_ref, o_ref, pooled_ref):
    i = pl.program_id(0)
    hout = _norm(aA_ref[...], aB_ref[...], h_ref[...], ws_ref[...],
                 dA_ref[...], dB_ref[...], b_ref[...])
    bcol = batch_ref[0]

    @pl.when(i == 0)
    def _():
        pooled_ref[...] = jnp.zeros((G, D), _f32)

    gmin = jnp.min(bcol)
    gmax = jnp.minimum(jnp.max(bcol), G - 1)

    def _pool_one(g, carry):
        red = jnp.max(jnp.where(bcol == g, hout, 0.0), axis=0, keepdims=True)
        pooled_ref[pl.ds(g, 1), :] = jnp.maximum(pooled_ref[pl.ds(g, 1), :],
                                                 red)
        return carry

    lax.fori_loop(gmin, gmax + 1, _pool_one, 0)

    @pl.when(i == NBLKP - 1)
    def _():
        o_ref[...] = (lax.dot_general(pooled_ref[...], wl_ref[...],
                                      (((1,), (0,)), ((), ())), **_DOT)
                      + bl_ref[...])


_k_final = pl.pallas_call(
    _final_body,
    grid=(NBLKP,),
    in_specs=[
        pl.BlockSpec((PROW, D), lambda i: (i, 0)),
        pl.BlockSpec((PROW, D), lambda i: (i, 0)),
        pl.BlockSpec((PROW, D), lambda i: (i, 0)),
        pl.BlockSpec((PROW, 1), lambda i: (i, 0)),
        pl.BlockSpec((PROW, 1), lambda i: (i, 0)),
        pl.BlockSpec((PROW, 1), lambda i: (i, 0)),
        pl.BlockSpec((1, D), lambda i: (0, 0)),
        pl.BlockSpec((1, PROW, 1), lambda i: (i, 0, 0)),
        pl.BlockSpec((D, 128), lambda i: (0, 0)),
        pl.BlockSpec((1, 128), lambda i: (0, 0)),
    ],
    out_specs=[pl.BlockSpec((G, 128), lambda i: (0, 0))],
    out_shape=[jax.ShapeDtypeStruct((G, 128), _f32)],
    scratch_shapes=[pltpu.VMEM((G, D), _f32)],
)


def kernel(x, edge_index, batch, W1, a_s1, a_d1, b1, W2, a_s2, a_d2, b2,
           Wl, bl):
    src = edge_index[0]
    dst = edge_index[1]
    pad = N + (jnp.arange(EPAD - E, dtype=jnp.int32) % (NPAD - N))
    srcp = jnp.concatenate([src, pad]).reshape(NW, NGRP, GRP, CHUNK)
    dstp = jnp.concatenate([dst, pad]).reshape(NW, NGRP, GRP, CHUNK)
    xp = jnp.zeros((NPAD, D), _f32).at[:N].set(x)
    batchp = jnp.concatenate(
        [batch, jnp.full((NPAD - N,), G, jnp.int32)]).reshape(NBLKP, PROW, 1)
    wlp = jnp.zeros((D, 128), _f32).at[:, :OUT].set(Wl)
    blp = jnp.zeros((1, 128), _f32).at[0, :OUT].set(bl)

    def pack_logits(sarr, darr):
        sb = jax.lax.bitcast_convert_type(
            sarr.reshape(NPAD).astype(jnp.bfloat16), jnp.uint16)
        db = jax.lax.bitcast_convert_type(
            darr.reshape(NPAD).astype(jnp.bfloat16), jnp.uint16)
        packed = (sb.astype(jnp.uint32) << 16) | db.astype(jnp.uint32)
        return jax.lax.bitcast_convert_type(packed, jnp.int32)

    h1, s1, d1, ws1 = _k_dense(xp, W1, a_s1.reshape(D, 1),
                               a_d1.reshape(D, 1))
    accA, accB, denA, denB = _sc_edges(h1, pack_logits(s1, d1), srcp, dstp)
    h2, s2, d2, ws2 = _k_norm_dense(
        accA, accB, h1, ws1, denA.reshape(NPAD, 1), denB.reshape(NPAD, 1),
        b1.reshape(1, D), W2, a_s2.reshape(D, 1), a_d2.reshape(D, 1))
    accA2, accB2, denA2, denB2 = _sc_edges(h2, pack_logits(s2, d2),
                                           srcp, dstp)
    o = _k_final(accA2, accB2, h2, ws2, denA2.reshape(NPAD, 1),
                 denB2.reshape(NPAD, 1), b2.reshape(1, D), batchp, wlp, blp)
    return o[0][:, :OUT]


# speedup vs baseline: 1.3262x; 1.0419x over previous
import dataclasses
import functools

import jax
import jax.numpy as jnp
from jax import lax
from jax.experimental import pallas as pl
from jax.experimental.pallas import tpu as pltpu
from jax.experimental.pallas import tpu_sc as plsc

N = 10000
NPAD = 10240
D = 128
E = 320000
G = 64
OUT = 5

NW = 32
CHUNK = 128
GRP = 8
NGRP = 10
CPT = GRP * NGRP
EPT = CPT * CHUNK
EPAD = NW * EPT
RPS = NPAD // 16
NBLK = NPAD // 128
BROW = 1024
NBLKD = NPAD // BROW
PROW = 512
NBLKP = NPAD // PROW

_mesh = plsc.VectorSubcoreMesh(core_axis_name="c", subcore_axis_name="s")

_sc_params = pltpu.CompilerParams()
if "needs_layout_passes" in pltpu.CompilerParams.__dataclass_fields__:
    _sc_params = dataclasses.replace(_sc_params, needs_layout_passes=False)

_f32 = jnp.float32
_DOT = dict(preferred_element_type=jnp.float32,
            precision=jax.lax.Precision.HIGHEST)


@functools.partial(
    pl.kernel,
    mesh=_mesh,
    compiler_params=_sc_params,
    out_type=[
        jax.ShapeDtypeStruct((NPAD, D), _f32),
        jax.ShapeDtypeStruct((NPAD, D), _f32),
        jax.ShapeDtypeStruct((NPAD,), _f32),
        jax.ShapeDtypeStruct((NPAD,), _f32),
    ],
    scratch_types=[
        pltpu.VMEM((NPAD,), jnp.int32),
        pltpu.VMEM((GRP, CHUNK), jnp.int32),
        pltpu.VMEM((GRP, CHUNK), jnp.int32),
        pltpu.VMEM((GRP, CHUNK), _f32),
        pltpu.VMEM((CHUNK, D), _f32),
        pltpu.VMEM((CHUNK, D), _f32),
        pltpu.SemaphoreType.DMA,
        pltpu.SemaphoreType.DMA,
        pltpu.SemaphoreType.DMA,
        pltpu.SemaphoreType.DMA,
        pltpu.SemaphoreType.DMA,
        pltpu.VMEM_SHARED((NPAD, D), _f32),
        pltpu.VMEM_SHARED((NPAD,), _f32),
    ],
)
def _sc_edges(h_hbm, t_hbm, src_hbm, dst_hbm,
              accA, accB, denA, denB,
              tv, srcg, dstg, wg, rows0, rows1,
              sem_w, sem_g0, sem_g1,
              sem_a0, sem_a1, accS, denS):
    c = lax.axis_index("c")
    s = lax.axis_index("s")
    wid = c * 16 + s
    rows = (rows0, rows1)
    sem_g = (sem_g0, sem_g1)
    sem_a = (sem_a0, sem_a1)

    @pl.loop(0, CHUNK)
    def _(r):
        for j in range(D // 16):
            rows0[r, pl.ds(j * 16, 16)] = jnp.zeros((16,), _f32)

    base = s * RPS
    for k in range(RPS // CHUNK):
        pltpu.sync_copy(rows0, accS.at[pl.ds(base + k * CHUNK, CHUNK)])
    rem = RPS % CHUNK
    if rem:
        pltpu.sync_copy(rows0.at[pl.ds(0, rem)],
                        accS.at[pl.ds(base + RPS - rem, rem)])
    for k in range(RPS // 128):
        pltpu.sync_copy(rows0.at[0], denS.at[pl.ds(base + k * 128, 128)])

    pltpu.sync_copy(t_hbm, tv)
    plsc.subcore_barrier()

    @pl.loop(0, NGRP)
    def _(g):
        pltpu.sync_copy(src_hbm.at[wid, g], srcg)
        pltpu.sync_copy(dst_hbm.at[wid, g], dstg)

        def fire(j, b):
            return pltpu.async_copy(h_hbm.at[srcg.at[j]], rows[b],
                                    sem_g[b])

        inflight = fire(0, 0)
        scat = [None, None]
        den_handles = []
        for j in range(GRP):
            b = j & 1
            inflight.wait()
            if j + 1 < GRP:
                if scat[1 - b] is not None:
                    scat[1 - b].wait()
                    scat[1 - b] = None
                inflight = fire(j + 1, 1 - b)
            for i in range(CHUNK // 16):
                sl = pl.ds(i * 16, 16)
                gs = plsc.load_gather(tv, [srcg[j, sl]])
                gd = plsc.load_gather(tv, [dstg[j, sl]])
                sval = plsc.bitcast(gs & jnp.int32(-65536), _f32)
                dval = plsc.bitcast(lax.shift_left(gd, 16), _f32)
                e = sval + dval
                wg[j, sl] = jnp.exp(jnp.maximum(e, 0.2 * e))
            den_handles.append(
                pltpu.async_copy(wg.at[j], denS.at[dstg.at[j]], sem_w,
                                 add=True))

            @pl.loop(0, CHUNK // 16)
            def _(i):
                wv = wg[j, pl.ds(i * 16, 16)]
                for l in range(16):
                    w = wv[l]
                    row = i * 16 + l
                    for jj in range(D // 16):
                        rows[b][row, pl.ds(jj * 16, 16)] = (
                            rows[b][row, pl.ds(jj * 16, 16)] * w)

            scat[b] = pltpu.async_copy(rows[b], accS.at[dstg.at[j]],
                                       sem_a[b], add=True)
        for hd in scat:
            if hd is not None:
                hd.wait()
        for hd in den_handles:
            hd.wait()

    plsc.subcore_barrier()

    @pl.when(c == 0)
    def _():
        for k in range(RPS // CHUNK):
            sl = pl.ds(s * RPS + k * CHUNK, CHUNK)
            pltpu.sync_copy(accS.at[sl], accA.at[sl])
        pltpu.sync_copy(denS.at[pl.ds(s * RPS, RPS)],
                        denA.at[pl.ds(s * RPS, RPS)])

    @pl.when(c == 1)
    def _():
        for k in range(RPS // CHUNK):
            sl = pl.ds(s * RPS + k * CHUNK, CHUNK)
            pltpu.sync_copy(accS.at[sl], accB.at[sl])
        pltpu.sync_copy(denS.at[pl.ds(s * RPS, RPS)],
                        denB.at[pl.ds(s * RPS, RPS)])


def _dense_body(x_ref, w_ref, as_ref, ad_ref, h_ref, s_ref, d_ref, ws_ref):
    h = lax.dot_general(x_ref[...], w_ref[...], (((1,), (0,)), ((), ())),
                        **_DOT)
    sv = lax.dot_general(h, as_ref[...], (((1,), (0,)), ((), ())), **_DOT)
    dv = lax.dot_general(h, ad_ref[...], (((1,), (0,)), ((), ())), **_DOT)
    t = sv + dv
    h_ref[...] = h
    s_ref[...] = sv
    d_ref[...] = dv
    ws_ref[...] = jnp.exp(jnp.maximum(t, 0.2 * t))


_k_dense = pl.pallas_call(
    _dense_body,
    grid=(NBLKD,),
    in_specs=[
        pl.BlockSpec((BROW, D), lambda i: (i, 0)),
        pl.BlockSpec((D, D), lambda i: (0, 0)),
        pl.BlockSpec((D, 1), lambda i: (0, 0)),
        pl.BlockSpec((D, 1), lambda i: (0, 0)),
    ],
    out_specs=[
        pl.BlockSpec((BROW, D), lambda i: (i, 0)),
        pl.BlockSpec((BROW, 1), lambda i: (i, 0)),
        pl.BlockSpec((BROW, 1), lambda i: (i, 0)),
        pl.BlockSpec((BROW, 1), lambda i: (i, 0)),
    ],
    out_shape=[
        jax.ShapeDtypeStruct((NPAD, D), _f32),
        jax.ShapeDtypeStruct((NPAD, 1), _f32),
        jax.ShapeDtypeStruct((NPAD, 1), _f32),
        jax.ShapeDtypeStruct((NPAD, 1), _f32),
    ],
)


def _norm(accA, accB, h, ws, dA, dB, b):
    den = dA + dB + ws + 1e-16
    acc = accA + accB + ws * h
    return jnp.maximum(acc / den + b, 0.0)


def _norm_dense_body(aA_ref, aB_ref, h_ref, ws_ref, dA_ref, dB_ref, b_ref,
                     w_ref, as_ref, ad_ref,
                     h2_ref, s_ref, d_ref, ws2_ref):
    x2 = _norm(aA_ref[...], aB_ref[...], h_ref[...], ws_ref[...],
               dA_ref[...], dB_ref[...], b_ref[...])
    h2 = lax.dot_general(x2, w_ref[...], (((1,), (0,)), ((), ())), **_DOT)
    sv = lax.dot_general(h2, as_ref[...], (((1,), (0,)), ((), ())), **_DOT)
    dv = lax.dot_general(h2, ad_ref[...], (((1,), (0,)), ((), ())), **_DOT)
    t = sv + dv
    h2_ref[...] = h2
    s_ref[...] = sv
    d_ref[...] = dv
    ws2_ref[...] = jnp.exp(jnp.maximum(t, 0.2 * t))


_k_norm_dense = pl.pallas_call(
    _norm_dense_body,
    grid=(NBLKD,),
    in_specs=[
        pl.BlockSpec((BROW, D), lambda i: (i, 0)),
        pl.BlockSpec((BROW, D), lambda i: (i, 0)),
        pl.BlockSpec((BROW, D), lambda i: (i, 0)),
        pl.BlockSpec((BROW, 1), lambda i: (i, 0)),
        pl.BlockSpec((BROW, 1), lambda i: (i, 0)),
        pl.BlockSpec((BROW, 1), lambda i: (i, 0)),
        pl.BlockSpec((1, D), lambda i: (0, 0)),
        pl.BlockSpec((D, D), lambda i: (0, 0)),
        pl.BlockSpec((D, 1), lambda i: (0, 0)),
        pl.BlockSpec((D, 1), lambda i: (0, 0)),
    ],
    out_specs=[
        pl.BlockSpec((BROW, D), lambda i: (i, 0)),
        pl.BlockSpec((BROW, 1), lambda i: (i, 0)),
        pl.BlockSpec((BROW, 1), lambda i: (i, 0)),
        pl.BlockSpec((BROW, 1), lambda i: (i, 0)),
    ],
    out_shape=[
        jax.ShapeDtypeStruct((NPAD, D), _f32),
        jax.ShapeDtypeStruct((NPAD, 1), _f32),
        jax.ShapeDtypeStruct((NPAD, 1), _f32),
        jax.ShapeDtypeStruct((NPAD, 1), _f32),
    ],
)


def _final_body(aA_ref, aB_ref, h_ref, ws_ref, dA_ref, dB_ref, b_ref,
                batch_ref, wl_ref, bl_ref, o_ref, pooled_ref):
    i = pl.program_id(0)
    hout = _norm(aA_ref[...], aB_ref[...], h_ref[...], ws_ref[...],
                 dA_ref[...], dB_ref[...], b_ref[...])
    bcol = batch_ref[0]

    @pl.when(i == 0)
    def _():
        pooled_ref[...] = jnp.zeros((G, D), _f32)

    gmin = jnp.min(bcol)
    gmax = jnp.minimum(jnp.max(bcol), G - 1)

    def _pool_one(g, carry):
        red = jnp.max(jnp.where(bcol == g, hout, 0.0), axis=0, keepdims=True)
        pooled_ref[pl.ds(g, 1), :] = jnp.maximum(pooled_ref[pl.ds(g, 1), :],
                                                 red)
        return carry

    lax.fori_loop(gmin, gmax + 1, _pool_one, 0)

    @pl.when(i == NBLKP - 1)
    def _():
        o_ref[...] = (lax.dot_general(pooled_ref[...], wl_ref[...],
                                      (((1,), (0,)), ((), ())), **_DOT)
                      + bl_ref[...])


_k_final = pl.pallas_call(
    _final_body,
    grid=(NBLKP,),
    in_specs=[
        pl.BlockSpec((PROW, D), lambda i: (i, 0)),
        pl.BlockSpec((PROW, D), lambda i: (i, 0)),
        pl.BlockSpec((PROW, D), lambda i: (i, 0)),
        pl.BlockSpec((PROW, 1), lambda i: (i, 0)),
        pl.BlockSpec((PROW, 1), lambda i: (i, 0)),
        pl.BlockSpec((PROW, 1), lambda i: (i, 0)),
        pl.BlockSpec((1, D), lambda i: (0, 0)),
        pl.BlockSpec((1, PROW, 1), lambda i: (i, 0, 0)),
        pl.BlockSpec((D, 128), lambda i: (0, 0)),
        pl.BlockSpec((1, 128), lambda i: (0, 0)),
    ],
    out_specs=[pl.BlockSpec((G, 128), lambda i: (0, 0))],
    out_shape=[jax.ShapeDtypeStruct((G, 128), _f32)],
    scratch_shapes=[pltpu.VMEM((G, D), _f32)],
)


def kernel(x, edge_index, batch, W1, a_s1, a_d1, b1, W2, a_s2, a_d2, b2,
           Wl, bl):
    src = edge_index[0]
    dst = edge_index[1]
    pad = N + (jnp.arange(EPAD - E, dtype=jnp.int32) % (NPAD - N))
    srcp = jnp.concatenate([src, pad]).reshape(NW, NGRP, GRP, CHUNK)
    dstp = jnp.concatenate([dst, pad]).reshape(NW, NGRP, GRP, CHUNK)
    xp = jnp.zeros((NPAD, D), _f32).at[:N].set(x)
    batchp = jnp.concatenate(
        [batch, jnp.full((NPAD - N,), G, jnp.int32)]).reshape(NBLKP, PROW, 1)
    wlp = jnp.zeros((D, 128), _f32).at[:, :OUT].set(Wl)
    blp = jnp.zeros((1, 128), _f32).at[0, :OUT].set(bl)

    def pack_logits(sarr, darr):
        sb = jax.lax.bitcast_convert_type(
            sarr.reshape(NPAD).astype(jnp.bfloat16), jnp.uint16)
        db = jax.lax.bitcast_convert_type(
            darr.reshape(NPAD).astype(jnp.bfloat16), jnp.uint16)
        packed = (sb.astype(jnp.uint32) << 16) | db.astype(jnp.uint32)
        return jax.lax.bitcast_convert_type(packed, jnp.int32)

    h1, s1, d1, ws1 = _k_dense(xp, W1, a_s1.reshape(D, 1),
                               a_d1.reshape(D, 1))
    accA, accB, denA, denB = _sc_edges(h1, pack_logits(s1, d1), srcp, dstp)
    h2, s2, d2, ws2 = _k_norm_dense(
        accA, accB, h1, ws1, denA.reshape(NPAD, 1), denB.reshape(NPAD, 1),
        b1.reshape(1, D), W2, a_s2.reshape(D, 1), a_d2.reshape(D, 1))
    accA2, accB2, denA2, denB2 = _sc_edges(h2, pack_logits(s2, d2),
                                           srcp, dstp)
    o = _k_final(accA2, accB2, h2, ws2, denA2.reshape(NPAD, 1),
                 denB2.reshape(NPAD, 1), b2.reshape(1, D), batchp, wlp, blp)
    return o[0][:, :OUT]
